# Initial kernel scaffold; baseline (speedup 1.0000x reference)
import jax
import jax.numpy as jnp
from jax.experimental import pallas as pl


def kernel(subword_embeddings, word_spans, w_score, b_score, Wg, bg, W1, b1, W2, b2, gamma, beta):
    raise NotImplementedError("write your pallas kernel here")


# speedup vs baseline: 1.9918x; 1.9918x over previous
import jax
import jax.numpy as jnp
from jax import lax
from jax.experimental import pallas as pl
from jax.experimental.pallas import tpu as pltpu

B, S, H, W, C = 8, 512, 1024, 256, 4096
CB = 1024
NC = C // CB


def _fused_kernel(starts_ref, ends_ref, x_ref, wrow_ref, wg_ref, bg_ref,
                  w1_ref, b1_ref, w2_ref, b2_ref, gamma_ref, beta_ref,
                  out_ref, fused_scr, acc_scr):
    c = pl.program_id(1)

    @pl.when(c == 0)
    def _compute_fused():
        x = x_ref[0]
        starts = starts_ref[0]
        ends = ends_ref[0]
        valid = (starts >= 0) & (ends > starts)
        iota = lax.broadcasted_iota(jnp.int32, (W, S), 1)
        pm = (iota >= starts) & (iota < ends) & valid
        pmf = pm.astype(jnp.float32)
        counts = jnp.maximum(jnp.sum(pmf, axis=1, keepdims=True), 1.0)
        pooled = jnp.dot(pmf, x, preferred_element_type=jnp.float32) / counts
        scores = jnp.sum(x * wrow_ref[...], axis=1)
        logits = jnp.where(pm, scores[None, :], -1e30)
        m = jnp.max(logits, axis=1, keepdims=True)
        e = jnp.exp(logits - m) * pmf
        z = jnp.maximum(jnp.sum(e, axis=1, keepdims=True), 1e-9)
        attended = jnp.dot(e / z, x, preferred_element_type=jnp.float32)
        g_lin = (jnp.dot(pooled, wg_ref[:H], preferred_element_type=jnp.float32)
                 + jnp.dot(attended, wg_ref[H:], preferred_element_type=jnp.float32)
                 + bg_ref[...])
        gate = jax.nn.sigmoid(g_lin)
        fused = gate * attended + (1.0 - gate) * pooled
        fused_scr[...] = fused
        acc_scr[...] = fused + b2_ref[...]

    fused = fused_scr[...]
    pre = jnp.dot(fused, w1_ref[...], preferred_element_type=jnp.float32) + b1_ref[...]
    h1 = 0.5 * pre * (1.0 + lax.erf(pre * 0.7071067811865476))
    acc_scr[...] += jnp.dot(h1, w2_ref[...], preferred_element_type=jnp.float32)

    @pl.when(c == NC - 1)
    def _finalize():
        acc = acc_scr[...]
        mu = jnp.mean(acc, axis=1, keepdims=True)
        var = jnp.mean((acc - mu) ** 2, axis=1, keepdims=True)
        out = (acc - mu) / jnp.sqrt(var + 1e-5) * gamma_ref[...] + beta_ref[...]
        starts = starts_ref[0]
        ends = ends_ref[0]
        validf = ((starts >= 0) & (ends > starts)).astype(jnp.float32)
        out_ref[0] = out * validf


def kernel(subword_embeddings, word_spans, w_score, b_score, Wg, bg, W1, b1, W2, b2, gamma, beta):
    x = subword_embeddings
    starts = word_spans[..., 0:1].astype(jnp.int32)
    ends = word_spans[..., 1:2].astype(jnp.int32)
    wrow = w_score.reshape(1, H)

    composed = pl.pallas_call(
        _fused_kernel,
        grid=(B, NC),
        in_specs=[
            pl.BlockSpec((1, W, 1), lambda b, c: (b, 0, 0)),
            pl.BlockSpec((1, W, 1), lambda b, c: (b, 0, 0)),
            pl.BlockSpec((1, S, H), lambda b, c: (b, 0, 0)),
            pl.BlockSpec((1, H), lambda b, c: (0, 0)),
            pl.BlockSpec((2 * H, H), lambda b, c: (0, 0)),
            pl.BlockSpec((1, H), lambda b, c: (0, 0)),
            pl.BlockSpec((H, CB), lambda b, c: (0, c)),
            pl.BlockSpec((1, CB), lambda b, c: (0, c)),
            pl.BlockSpec((CB, H), lambda b, c: (c, 0)),
            pl.BlockSpec((1, H), lambda b, c: (0, 0)),
            pl.BlockSpec((1, H), lambda b, c: (0, 0)),
            pl.BlockSpec((1, H), lambda b, c: (0, 0)),
        ],
        out_specs=pl.BlockSpec((1, W, H), lambda b, c: (b, 0, 0)),
        out_shape=jax.ShapeDtypeStruct((B, W, H), jnp.float32),
        scratch_shapes=[pltpu.VMEM((W, H), jnp.float32),
                        pltpu.VMEM((W, H), jnp.float32)],
        compiler_params=pltpu.CompilerParams(
            dimension_semantics=("parallel", "arbitrary")),
    )(starts, ends, x, wrow, Wg, bg.reshape(1, H), W1, b1.reshape(1, C),
      W2, b2.reshape(1, H), gamma.reshape(1, H), beta.reshape(1, H))

    start = word_spans[..., 0]
    end = word_spans[..., 1]
    valid = (start >= 0) & (end > start)
    index = jnp.where(valid, start, -1)
    return composed, valid, index


# speedup vs baseline: 2.8878x; 1.4498x over previous
import jax
import jax.numpy as jnp
from jax import lax
from jax.experimental import pallas as pl
from jax.experimental.pallas import tpu as pltpu

B, S, H, W, C = 8, 512, 1024, 256, 4096
BW = B * W
CB = 512
NC = C // CB


def _fuse_kernel(starts_ref, ends_ref, x_ref, wrow_ref, wg_ref, bg_ref,
                 fused_ref):
    x = x_ref[0]
    starts = starts_ref[0]
    ends = ends_ref[0]
    valid = (starts >= 0) & (ends > starts)
    iota = lax.broadcasted_iota(jnp.int32, (W, S), 1)
    pm = (iota >= starts) & (iota < ends) & valid
    pmf = pm.astype(jnp.float32)
    counts = jnp.maximum(jnp.sum(pmf, axis=1, keepdims=True), 1.0)
    scores = jnp.sum(x * wrow_ref[...], axis=1)
    logits = jnp.where(pm, scores[None, :], -1e30)
    m = jnp.max(logits, axis=1, keepdims=True)
    e = jnp.exp(logits - m) * pmf
    z = jnp.maximum(jnp.sum(e, axis=1, keepdims=True), 1e-9)
    coef = jnp.concatenate([pmf / counts, e / z], axis=0)
    pa = jnp.dot(coef, x, preferred_element_type=jnp.float32)
    pooled = pa[:W]
    attended = pa[W:]
    g_in = jnp.concatenate([pooled, attended], axis=1)
    gate = jax.nn.sigmoid(
        jnp.dot(g_in, wg_ref[...], preferred_element_type=jnp.float32)
        + bg_ref[...])
    fused_ref[0] = gate * attended + (1.0 - gate) * pooled


def _mlp_kernel(fused_ref, w1_ref, b1_ref, w2_ref, b2_ref, gamma_ref,
                beta_ref, starts_ref, ends_ref, out_ref, acc_scr):
    c = pl.program_id(0)

    @pl.when(c == 0)
    def _init():
        acc_scr[...] = fused_ref[...] + b2_ref[...]

    pre = jnp.dot(fused_ref[...], w1_ref[...],
                  preferred_element_type=jnp.float32) + b1_ref[...]
    h1 = 0.5 * pre * (1.0 + lax.erf(pre * 0.7071067811865476))
    acc_scr[...] += jnp.dot(h1, w2_ref[...], preferred_element_type=jnp.float32)

    @pl.when(c == NC - 1)
    def _finalize():
        acc = acc_scr[...]
        mu = jnp.mean(acc, axis=1, keepdims=True)
        var = jnp.mean((acc - mu) ** 2, axis=1, keepdims=True)
        out = (acc - mu) / jnp.sqrt(var + 1e-5) * gamma_ref[...] + beta_ref[...]
        starts = starts_ref[...]
        ends = ends_ref[...]
        validf = ((starts >= 0) & (ends > starts)).astype(jnp.float32)
        out_ref[...] = out * validf


def kernel(subword_embeddings, word_spans, w_score, b_score, Wg, bg, W1, b1, W2, b2, gamma, beta):
    x = subword_embeddings
    starts = word_spans[..., 0:1].astype(jnp.int32)
    ends = word_spans[..., 1:2].astype(jnp.int32)
    wrow = w_score.reshape(1, H)

    fused = pl.pallas_call(
        _fuse_kernel,
        grid=(B,),
        in_specs=[
            pl.BlockSpec((1, W, 1), lambda b: (b, 0, 0)),
            pl.BlockSpec((1, W, 1), lambda b: (b, 0, 0)),
            pl.BlockSpec((1, S, H), lambda b: (b, 0, 0)),
            pl.BlockSpec((1, H), lambda b: (0, 0)),
            pl.BlockSpec((2 * H, H), lambda b: (0, 0)),
            pl.BlockSpec((1, H), lambda b: (0, 0)),
        ],
        out_specs=pl.BlockSpec((1, W, H), lambda b: (b, 0, 0)),
        out_shape=jax.ShapeDtypeStruct((B, W, H), jnp.float32),
        compiler_params=pltpu.CompilerParams(
            dimension_semantics=("arbitrary",)),
    )(starts, ends, x, wrow, Wg, bg.reshape(1, H))

    composed = pl.pallas_call(
        _mlp_kernel,
        grid=(NC,),
        in_specs=[
            pl.BlockSpec((BW, H), lambda c: (0, 0)),
            pl.BlockSpec((H, CB), lambda c: (0, c)),
            pl.BlockSpec((1, CB), lambda c: (0, c)),
            pl.BlockSpec((CB, H), lambda c: (c, 0)),
            pl.BlockSpec((1, H), lambda c: (0, 0)),
            pl.BlockSpec((1, H), lambda c: (0, 0)),
            pl.BlockSpec((1, H), lambda c: (0, 0)),
            pl.BlockSpec((BW, 1), lambda c: (0, 0)),
            pl.BlockSpec((BW, 1), lambda c: (0, 0)),
        ],
        out_specs=pl.BlockSpec((BW, H), lambda c: (0, 0)),
        out_shape=jax.ShapeDtypeStruct((BW, H), jnp.float32),
        scratch_shapes=[pltpu.VMEM((BW, H), jnp.float32)],
        compiler_params=pltpu.CompilerParams(
            dimension_semantics=("arbitrary",)),
    )(fused.reshape(BW, H), W1, b1.reshape(1, C), W2, b2.reshape(1, H),
      gamma.reshape(1, H), beta.reshape(1, H),
      starts.reshape(BW, 1), ends.reshape(BW, 1))

    composed = composed.reshape(B, W, H)
    start = word_spans[..., 0]
    end = word_spans[..., 1]
    valid = (start >= 0) & (end > start)
    index = jnp.where(valid, start, -1)
    return composed, valid, index
